# baseline (speedup 1.0000x reference)
import jax
import jax.numpy as jnp
from jax.experimental import pallas as pl
from jax.experimental.pallas import tpu as pltpu

_C = 64
_B = 25000


def _body(x_ref, w_ref, t_ref, loss_ref, acc_ref):
    i = pl.program_id(0)
    nb = pl.num_programs(0)

    @pl.when(i == 0)
    def _init():
        acc_ref[...] = jnp.zeros_like(acc_ref)
        loss_ref[...] = jnp.zeros_like(loss_ref)

    x = x_ref[...]
    w = w_ref[...]
    scaled = x * w
    rowmax = jnp.max(scaled, axis=1, keepdims=True)
    mask = (scaled == rowmax).astype(jnp.bfloat16)
    r2 = jax.lax.broadcasted_iota(jnp.int32, (_C, _C), 0)
    c2 = jax.lax.broadcasted_iota(jnp.int32, (_C, _C), 1)
    lower_tri = (r2 < c2).astype(jnp.bfloat16)
    prefix = jax.lax.dot_general(
        mask, lower_tri, (((1,), (0,)), ((), ())),
        preferred_element_type=jnp.float32)
    oh_pred = jnp.where(prefix == 0.0, mask, jnp.bfloat16(0.0))
    t_row = t_ref[0]
    cls_i = jax.lax.broadcasted_iota(jnp.int32, (_C, _B), 0)
    oh_tgt_t = (cls_i == t_row).astype(jnp.bfloat16)
    acc_ref[...] += jax.lax.dot_general(
        oh_tgt_t, oh_pred, (((1,), (0,)), ((), ())),
        preferred_element_type=jnp.float32)

    @pl.when(i == nb - 1)
    def _epilogue():
        cmt = acc_ref[...]
        r_iota = jax.lax.broadcasted_iota(jnp.int32, (_C, _C), 0)
        c_iota = jax.lax.broadcasted_iota(jnp.int32, (_C, _C), 1)
        eye = (r_iota == c_iota).astype(jnp.float32)
        colsum = jnp.sum(cmt, axis=0, keepdims=True)
        rowsum = jnp.sum(cmt, axis=1, keepdims=True)
        diag_row = jnp.sum(cmt * eye, axis=0, keepdims=True)
        diag_col = jnp.sum(cmt * eye, axis=1, keepdims=True)
        p = diag_col / rowsum
        r = diag_row / colsum
        f1 = 2.0 * p * r / (p + r)
        f1_diag = jnp.where(r_iota == c_iota, f1, 0.0)
        loss_ref[...] = -jnp.sum(f1_diag, axis=(0, 1), keepdims=True) / _C


def kernel(inputs, targets, class_weight):
    n = inputs.shape[0]
    nb = n // _B
    w2 = class_weight.reshape(1, _C)
    t3 = targets.reshape(nb, 1, _B)
    loss = pl.pallas_call(
        _body,
        grid=(nb,),
        in_specs=[
            pl.BlockSpec((_B, _C), lambda i: (i, 0)),
            pl.BlockSpec((1, _C), lambda i: (0, 0)),
            pl.BlockSpec((1, 1, _B), lambda i: (i, 0, 0)),
        ],
        out_specs=pl.BlockSpec((1, 1), lambda i: (0, 0)),
        out_shape=jax.ShapeDtypeStruct((1, 1), jnp.float32),
        scratch_shapes=[pltpu.VMEM((_C, _C), jnp.float32)],
    )(inputs, w2, t3)
    return (loss.reshape(()), class_weight)


# speedup vs baseline: 1.0336x; 1.0336x over previous
import jax
import jax.numpy as jnp
from jax.experimental import pallas as pl
from jax.experimental.pallas import tpu as pltpu

_C = 64
_B = 20000


def _body(x_ref, w_ref, t_ref, loss_ref, acc_ref):
    i = pl.program_id(0)
    nb = pl.num_programs(0)

    @pl.when(i == 0)
    def _init():
        acc_ref[...] = jnp.zeros_like(acc_ref)
        loss_ref[...] = jnp.zeros_like(loss_ref)

    x = x_ref[...]
    w = w_ref[...]
    scaled = x * w
    rowmax = jnp.max(scaled, axis=1, keepdims=True)
    mask = (scaled == rowmax).astype(jnp.bfloat16)
    r2 = jax.lax.broadcasted_iota(jnp.int32, (_C, _C), 0)
    c2 = jax.lax.broadcasted_iota(jnp.int32, (_C, _C), 1)
    lower_tri = (r2 < c2).astype(jnp.bfloat16)
    prefix = jax.lax.dot_general(
        mask, lower_tri, (((1,), (0,)), ((), ())),
        preferred_element_type=jnp.float32)
    oh_pred = jnp.where(prefix == 0.0, mask, jnp.bfloat16(0.0))
    t_row = t_ref[0]
    cls_i = jax.lax.broadcasted_iota(jnp.int32, (_C, _B), 0)
    oh_tgt_t = (cls_i == t_row).astype(jnp.bfloat16)
    acc_ref[...] += jax.lax.dot_general(
        oh_tgt_t, oh_pred, (((1,), (0,)), ((), ())),
        preferred_element_type=jnp.float32)

    @pl.when(i == nb - 1)
    def _epilogue():
        cmt = acc_ref[...]
        r_iota = jax.lax.broadcasted_iota(jnp.int32, (_C, _C), 0)
        c_iota = jax.lax.broadcasted_iota(jnp.int32, (_C, _C), 1)
        eye = (r_iota == c_iota).astype(jnp.float32)
        colsum = jnp.sum(cmt, axis=0, keepdims=True)
        rowsum = jnp.sum(cmt, axis=1, keepdims=True)
        diag_row = jnp.sum(cmt * eye, axis=0, keepdims=True)
        diag_col = jnp.sum(cmt * eye, axis=1, keepdims=True)
        p = diag_col / rowsum
        r = diag_row / colsum
        f1 = 2.0 * p * r / (p + r)
        f1_diag = jnp.where(r_iota == c_iota, f1, 0.0)
        loss_ref[...] = -jnp.sum(f1_diag, axis=(0, 1), keepdims=True) / _C


def kernel(inputs, targets, class_weight):
    n = inputs.shape[0]
    nb = n // _B
    w2 = class_weight.reshape(1, _C)
    t3 = targets.reshape(nb, 1, _B)
    loss = pl.pallas_call(
        _body,
        grid=(nb,),
        in_specs=[
            pl.BlockSpec((_B, _C), lambda i: (i, 0)),
            pl.BlockSpec((1, _C), lambda i: (0, 0)),
            pl.BlockSpec((1, 1, _B), lambda i: (i, 0, 0)),
        ],
        out_specs=pl.BlockSpec((1, 1), lambda i: (0, 0)),
        out_shape=jax.ShapeDtypeStruct((1, 1), jnp.float32),
        scratch_shapes=[pltpu.VMEM((_C, _C), jnp.float32)],
    )(inputs, w2, t3)
    return (loss.reshape(()), class_weight)
